# baseline (speedup 1.0000x reference)
import jax
import jax.numpy as jnp
from jax import lax
from jax.experimental import pallas as pl
from jax.experimental.pallas import tpu as pltpu
from jax.experimental.pallas import tpu_sc as plsc

N = 10000
E = 320000
IN_C = 128
HID = 256
H2 = HID // 2
NS = 16
NP = 10240
NPT = NP // NS
K = 125
CH = E // (K * NS)
NBG = 16
NG = CH // NBG
R = 2000
G = N // R
_mesh = plsc.VectorSubcoreMesh(core_axis_name="c", subcore_axis_name="s")


HCH = E // (32 * K)


def _hist_body(colm, out, acc_sp, idx_v, ones_v, zeros_v, sem):
    c = lax.axis_index("c")
    s = lax.axis_index("s")
    w = c * NS + s
    pltpu.sync_copy(colm.at[pl.ds(w * HCH, HCH)], idx_v)

    def fill(i, carry):
        ones_v[pl.ds(i * 16, 16)] = jnp.full((16,), 1.0, jnp.float32)
        zeros_v[pl.ds(i * 16, 16)] = jnp.zeros((16,), jnp.float32)
        return carry
    lax.fori_loop(0, NPT // 16, fill, 0)

    r0 = s * NPT
    pltpu.sync_copy(zeros_v, acc_sp.at[pl.ds(r0, NPT)])
    plsc.subcore_barrier()

    def step(i, carry):
        pltpu.sync_copy(ones_v.at[pl.ds(0, K)],
                        acc_sp.at[idx_v.at[i]], add=True)
        return carry
    lax.fori_loop(0, HCH, step, 0)
    plsc.subcore_barrier()

    @pl.when(c == 0)
    def _():
        pltpu.sync_copy(acc_sp.at[pl.ds(r0, NPT)], out.at[0, 0, pl.ds(r0, NPT)])

    @pl.when(c == 1)
    def _():
        pltpu.sync_copy(acc_sp.at[pl.ds(r0, NPT)], out.at[1, 0, pl.ds(r0, NPT)])


_hist_call = pl.kernel(
    _hist_body,
    out_type=jax.ShapeDtypeStruct((2, 1, NP), jnp.float32),
    mesh=_mesh,
    scratch_types=[
        pltpu.VMEM_SHARED((NP,), jnp.float32),
        pltpu.VMEM((HCH, K), jnp.int32),
        pltpu.VMEM((NPT,), jnp.float32),
        pltpu.VMEM((NPT,), jnp.float32),
        pltpu.SemaphoreType.DMA,
    ],
)


def _pass_body(y0, y1, rowm, colm, out0, out1, acc,
               row_v, col_v, g0, g1, sg0, sg1, ss0, ss1):
    c = lax.axis_index("c")
    s = lax.axis_index("s")

    r0 = s * NPT

    @pl.when(c == 0)
    def _():
        pltpu.sync_copy(y0.at[pl.ds(r0, NPT)], acc.at[pl.ds(r0, NPT)])

    @pl.when(c == 1)
    def _():
        pltpu.sync_copy(y1.at[pl.ds(r0, NPT)], acc.at[pl.ds(r0, NPT)])

    plsc.subcore_barrier()

    gbufs = (g0, g1)
    gsems = (sg0, sg1)
    ssems = (ss0, ss1)

    def start_gather(idx_view, buf, sem):
        @pl.when(c == 0)
        def _():
            pltpu.async_copy(y0.at[idx_view], buf, sem)

        @pl.when(c == 1)
        def _():
            pltpu.async_copy(y1.at[idx_view], buf, sem)

    def wait_gather(buf, sem):
        pltpu.make_async_copy(y0.at[row_v.at[0]], buf, sem).wait()

    def wait_scatter(buf, sem):
        pltpu.make_async_copy(buf, acc.at[col_v.at[0]], sem).wait()

    def group(g, carry):
        off = pl.multiple_of((s * NG + g) * NBG, 8)
        pltpu.sync_copy(rowm.at[pl.ds(off, NBG)], row_v)
        pltpu.sync_copy(colm.at[pl.ds(off, NBG)], col_v)
        for k in range(NBG):
            p = k % 2
            q = (k + 1) % 2
            if k == 0:
                @pl.when(g > 0)
                def _():
                    wait_scatter(gbufs[0], ssems[0])
                start_gather(row_v.at[0], gbufs[0], gsems[0])
            if k + 1 < NBG:
                if k + 1 == 1:
                    @pl.when(g > 0)
                    def _():
                        wait_scatter(gbufs[1], ssems[1])
                else:
                    wait_scatter(gbufs[q], ssems[q])
                start_gather(row_v.at[k + 1], gbufs[q], gsems[q])
            wait_gather(gbufs[p], gsems[p])
            pltpu.async_copy(gbufs[p], acc.at[col_v.at[k]], ssems[p], add=True)
        return carry

    lax.fori_loop(0, NG, group, 0)
    wait_scatter(gbufs[0], ssems[0])
    wait_scatter(gbufs[1], ssems[1])
    plsc.subcore_barrier()

    @pl.when(c == 0)
    def _():
        pltpu.sync_copy(acc.at[pl.ds(r0, NPT)], out0.at[pl.ds(r0, NPT)])

    @pl.when(c == 1)
    def _():
        pltpu.sync_copy(acc.at[pl.ds(r0, NPT)], out1.at[pl.ds(r0, NPT)])


_pass_call = pl.kernel(
    _pass_body,
    out_type=[jax.ShapeDtypeStruct((NP, H2), jnp.float32),
              jax.ShapeDtypeStruct((NP, H2), jnp.float32)],
    mesh=_mesh,
    scratch_types=[
        pltpu.VMEM_SHARED((NP, H2), jnp.float32),
        pltpu.VMEM((NBG, K), jnp.int32),
        pltpu.VMEM((NBG, K), jnp.int32),
        pltpu.VMEM((K, H2), jnp.float32),
        pltpu.VMEM((K, H2), jnp.float32),
        pltpu.SemaphoreType.DMA,
        pltpu.SemaphoreType.DMA,
        pltpu.SemaphoreType.DMA,
        pltpu.SemaphoreType.DMA,
    ],
)


def _deg_spec():
    return pl.BlockSpec((2, R, 1), lambda i: (0, i, 0))


def _dvec(h_blk):
    return lax.rsqrt(h_blk[0] + h_blk[1] + 1.0)


_DNUMS = (((1,), (1,)), ((), ()))


def _matmul1(x, w1, b1, deg):
    def kern(x_r, w_r, b_r, h_r, xl_r, y0_r, y1_r):
        xl = lax.dot_general(x_r[...], w_r[...], _DNUMS,
                             preferred_element_type=jnp.float32) + b_r[...]
        xl_r[...] = xl
        y = _dvec(h_r[...]) * xl
        y0_r[...] = y[:, :H2]
        y1_r[...] = y[:, H2:]

    return pl.pallas_call(
        kern,
        grid=(G,),
        in_specs=[
            pl.BlockSpec((R, IN_C), lambda i: (i, 0)),
            pl.BlockSpec((HID, IN_C), lambda i: (0, 0)),
            pl.BlockSpec((1, HID), lambda i: (0, 0)),
            _deg_spec(),
        ],
        out_specs=[
            pl.BlockSpec((R, HID), lambda i: (i, 0)),
            pl.BlockSpec((R, H2), lambda i: (i, 0)),
            pl.BlockSpec((R, H2), lambda i: (i, 0)),
        ],
        out_shape=[
            jax.ShapeDtypeStruct((N, HID), jnp.float32),
            jax.ShapeDtypeStruct((NP, H2), jnp.float32),
            jax.ShapeDtypeStruct((NP, H2), jnp.float32),
        ],
    )(x, w1, b1, deg)


def _combine_mid(a0, a1, xl, deg, m):
    def kern(a0_r, a1_r, xl_r, h_r, m_r, y0_r, y1_r):
        dd = _dvec(h_r[...])
        mm = m_r[...]
        xl = xl_r[...]
        y0_r[...] = dd * jnp.where(mm, dd * a0_r[...], xl[:, :H2])
        y1_r[...] = dd * jnp.where(mm, dd * a1_r[...], xl[:, H2:])

    return pl.pallas_call(
        kern,
        grid=(G,),
        in_specs=[
            pl.BlockSpec((R, H2), lambda i: (i, 0)),
            pl.BlockSpec((R, H2), lambda i: (i, 0)),
            pl.BlockSpec((R, HID), lambda i: (i, 0)),
            _deg_spec(),
            pl.BlockSpec((R, 1), lambda i: (i, 0)),
        ],
        out_specs=[
            pl.BlockSpec((R, H2), lambda i: (i, 0)),
            pl.BlockSpec((R, H2), lambda i: (i, 0)),
        ],
        out_shape=[
            jax.ShapeDtypeStruct((NP, H2), jnp.float32),
            jax.ShapeDtypeStruct((NP, H2), jnp.float32),
        ],
    )(a0, a1, xl, deg, m)


def _layer2_head(a0, a1, xl1, deg, m, bias1, w2, b2):
    def kern(a0_r, a1_r, xl_r, h_r, m_r, b_r, w_r, b2_r, xl2_r, z0_r, z1_r):
        dd = _dvec(h_r[...])
        mm = m_r[...]
        xl = xl_r[...]
        b = b_r[...]
        t0 = jnp.where(mm, xl[:, :H2], dd * a0_r[...]) + b[:, :H2]
        t1 = jnp.where(mm, xl[:, H2:], dd * a1_r[...]) + b[:, H2:]
        t0 = jnp.maximum(t0, 0.0)
        t1 = jnp.maximum(t1, 0.0)
        w = w_r[...]
        xl2 = (lax.dot_general(t0, w[:, :H2], _DNUMS,
                               preferred_element_type=jnp.float32)
               + lax.dot_general(t1, w[:, H2:], _DNUMS,
                                 preferred_element_type=jnp.float32)
               + b2_r[...])
        xl2_r[...] = xl2
        y = dd * xl2
        z0_r[...] = y[:, :H2]
        z1_r[...] = y[:, H2:]

    return pl.pallas_call(
        kern,
        grid=(G,),
        in_specs=[
            pl.BlockSpec((R, H2), lambda i: (i, 0)),
            pl.BlockSpec((R, H2), lambda i: (i, 0)),
            pl.BlockSpec((R, HID), lambda i: (i, 0)),
            _deg_spec(),
            pl.BlockSpec((R, 1), lambda i: (i, 0)),
            pl.BlockSpec((1, HID), lambda i: (0, 0)),
            pl.BlockSpec((HID, HID), lambda i: (0, 0)),
            pl.BlockSpec((1, HID), lambda i: (0, 0)),
        ],
        out_specs=[
            pl.BlockSpec((R, HID), lambda i: (i, 0)),
            pl.BlockSpec((R, H2), lambda i: (i, 0)),
            pl.BlockSpec((R, H2), lambda i: (i, 0)),
        ],
        out_shape=[
            jax.ShapeDtypeStruct((N, HID), jnp.float32),
            jax.ShapeDtypeStruct((NP, H2), jnp.float32),
            jax.ShapeDtypeStruct((NP, H2), jnp.float32),
        ],
    )(a0, a1, xl1, deg, m, bias1, w2, b2)


def _combine_final(a0, a1, xl, deg, m, bias):
    def kern(a0_r, a1_r, xl_r, h_r, m_r, b_r, o_r):
        dd = _dvec(h_r[...])
        mm = m_r[...]
        xl = xl_r[...]
        b = b_r[...]
        t0 = jnp.where(mm, xl[:, :H2], dd * a0_r[...]) + b[:, :H2]
        t1 = jnp.where(mm, xl[:, H2:], dd * a1_r[...]) + b[:, H2:]
        o_r[...] = jnp.concatenate([t0, t1], axis=1)

    return pl.pallas_call(
        kern,
        grid=(G,),
        in_specs=[
            pl.BlockSpec((R, H2), lambda i: (i, 0)),
            pl.BlockSpec((R, H2), lambda i: (i, 0)),
            pl.BlockSpec((R, HID), lambda i: (i, 0)),
            _deg_spec(),
            pl.BlockSpec((R, 1), lambda i: (i, 0)),
            pl.BlockSpec((1, HID), lambda i: (0, 0)),
        ],
        out_specs=pl.BlockSpec((R, HID), lambda i: (i, 0)),
        out_shape=jax.ShapeDtypeStruct((N, HID), jnp.float32),
    )(a0, a1, xl, deg, m, bias)


def kernel(x, edge_index, M, W1, b1, bias1, W2, b2, bias2):
    row = edge_index[0]
    col = edge_index[1]
    rowm = row.reshape(E // K, K)
    colm = col.reshape(E // K, K)
    deg = _hist_call(colm).reshape(2, NP, 1)

    b1r = b1.reshape(1, HID)
    b2r = b2.reshape(1, HID)
    bias1r = bias1.reshape(1, HID)
    bias2r = bias2.reshape(1, HID)

    xl1, y0, y1 = _matmul1(x, W1, b1r, deg)
    a0, a1 = _pass_call(y0, y1, rowm, colm)
    y0b, y1b = _combine_mid(a0, a1, xl1, deg, M)
    a0b, a1b = _pass_call(y0b, y1b, rowm, colm)

    xl2, z0, z1 = _layer2_head(a0b, a1b, xl1, deg, M, bias1r, W2, b2r)

    c0, c1 = _pass_call(z0, z1, rowm, colm)
    z0b, z1b = _combine_mid(c0, c1, xl2, deg, M)
    c0b, c1b = _pass_call(z0b, z1b, rowm, colm)
    return _combine_final(c0b, c1b, xl2, deg, M, bias2r)


# speedup vs baseline: 1.0177x; 1.0177x over previous
import jax
import jax.numpy as jnp
from jax import lax
from jax.experimental import pallas as pl
from jax.experimental.pallas import tpu as pltpu
from jax.experimental.pallas import tpu_sc as plsc

N = 10000
E = 320000
IN_C = 128
HID = 256
H2 = HID // 2
NS = 16
NP = 10240
NPT = NP // NS
K = 125
CH = E // (K * NS)
NBG = 16
NG = CH // NBG
R = 2000
G = N // R
_mesh = plsc.VectorSubcoreMesh(core_axis_name="c", subcore_axis_name="s")


HCH = E // (32 * K)


def _hist_body(e3, out, acc_sp, idx_v, ones_v, zeros_v, sem):
    c = lax.axis_index("c")
    s = lax.axis_index("s")
    w = c * NS + s
    pltpu.sync_copy(e3.at[1, pl.ds(w * HCH, HCH)], idx_v)

    def fill(i, carry):
        ones_v[pl.ds(i * 16, 16)] = jnp.full((16,), 1.0, jnp.float32)
        zeros_v[pl.ds(i * 16, 16)] = jnp.zeros((16,), jnp.float32)
        return carry
    lax.fori_loop(0, NPT // 16, fill, 0)

    r0 = s * NPT
    pltpu.sync_copy(zeros_v, acc_sp.at[pl.ds(r0, NPT)])
    plsc.subcore_barrier()

    def step(i, carry):
        pltpu.sync_copy(ones_v.at[pl.ds(0, K)],
                        acc_sp.at[idx_v.at[i]], add=True)
        return carry
    lax.fori_loop(0, HCH, step, 0)
    plsc.subcore_barrier()

    @pl.when(c == 0)
    def _():
        pltpu.sync_copy(acc_sp.at[pl.ds(r0, NPT)], out.at[0, 0, pl.ds(r0, NPT)])

    @pl.when(c == 1)
    def _():
        pltpu.sync_copy(acc_sp.at[pl.ds(r0, NPT)], out.at[1, 0, pl.ds(r0, NPT)])


_hist_call = pl.kernel(
    _hist_body,
    out_type=jax.ShapeDtypeStruct((2, 1, NP), jnp.float32),
    mesh=_mesh,
    scratch_types=[
        pltpu.VMEM_SHARED((NP,), jnp.float32),
        pltpu.VMEM((HCH, K), jnp.int32),
        pltpu.VMEM((NPT,), jnp.float32),
        pltpu.VMEM((NPT,), jnp.float32),
        pltpu.SemaphoreType.DMA,
    ],
)


def _pass_body(y0, y1, e3, out0, out1, acc,
               row_v, col_v, g0, g1, sg0, sg1, ss0, ss1):
    c = lax.axis_index("c")
    s = lax.axis_index("s")

    r0 = s * NPT

    off0 = pl.multiple_of(s * CH, 8)
    pltpu.sync_copy(e3.at[0, pl.ds(off0, NBG)], row_v.at[0])
    pltpu.sync_copy(e3.at[1, pl.ds(off0, NBG)], col_v.at[0])

    def start_gather(idx_view, buf, sem):
        @pl.when(c == 0)
        def _():
            pltpu.async_copy(y0.at[idx_view], buf, sem)

        @pl.when(c == 1)
        def _():
            pltpu.async_copy(y1.at[idx_view], buf, sem)

    start_gather(row_v.at[0, 0], g0, sg0)

    @pl.when(c == 0)
    def _():
        pltpu.sync_copy(y0.at[pl.ds(r0, NPT)], acc.at[pl.ds(r0, NPT)])

    @pl.when(c == 1)
    def _():
        pltpu.sync_copy(y1.at[pl.ds(r0, NPT)], acc.at[pl.ds(r0, NPT)])

    plsc.subcore_barrier()

    gbufs = (g0, g1)
    gsems = (sg0, sg1)
    ssems = (ss0, ss1)

    def wait_gather(buf, sem):
        pltpu.make_async_copy(y0.at[row_v.at[0, 0]], buf, sem).wait()

    def wait_scatter(buf, sem):
        pltpu.make_async_copy(buf, acc.at[col_v.at[0, 0]], sem).wait()

    def group(g, carry):
        slot = g & 1
        @pl.when(g > 0)
        def _():
            off = pl.multiple_of((s * NG + g) * NBG, 8)
            pltpu.sync_copy(e3.at[0, pl.ds(off, NBG)], row_v.at[slot])
            pltpu.sync_copy(e3.at[1, pl.ds(off, NBG)], col_v.at[slot])
        for k in range(NBG):
            p = k % 2
            q = (k + 1) % 2
            if k == 0:
                @pl.when(g > 0)
                def _():
                    wait_scatter(gbufs[0], ssems[0])
                    start_gather(row_v.at[slot, 0], gbufs[0], gsems[0])
            if k + 1 < NBG:
                if k + 1 == 1:
                    @pl.when(g > 0)
                    def _():
                        wait_scatter(gbufs[1], ssems[1])
                else:
                    wait_scatter(gbufs[q], ssems[q])
                start_gather(row_v.at[slot, k + 1], gbufs[q], gsems[q])
            wait_gather(gbufs[p], gsems[p])
            pltpu.async_copy(gbufs[p], acc.at[col_v.at[slot, k]],
                             ssems[p], add=True)
        return carry

    lax.fori_loop(0, NG, group, 0)
    wait_scatter(gbufs[0], ssems[0])
    wait_scatter(gbufs[1], ssems[1])
    plsc.subcore_barrier()

    @pl.when(c == 0)
    def _():
        pltpu.sync_copy(acc.at[pl.ds(r0, NPT)], out0.at[pl.ds(r0, NPT)])

    @pl.when(c == 1)
    def _():
        pltpu.sync_copy(acc.at[pl.ds(r0, NPT)], out1.at[pl.ds(r0, NPT)])


_pass_call = pl.kernel(
    _pass_body,
    out_type=[jax.ShapeDtypeStruct((NP, H2), jnp.float32),
              jax.ShapeDtypeStruct((NP, H2), jnp.float32)],
    mesh=_mesh,
    scratch_types=[
        pltpu.VMEM_SHARED((NP, H2), jnp.float32),
        pltpu.VMEM((2, NBG, K), jnp.int32),
        pltpu.VMEM((2, NBG, K), jnp.int32),
        pltpu.VMEM((K, H2), jnp.float32),
        pltpu.VMEM((K, H2), jnp.float32),
        pltpu.SemaphoreType.DMA,
        pltpu.SemaphoreType.DMA,
        pltpu.SemaphoreType.DMA,
        pltpu.SemaphoreType.DMA,
    ],
)


def _deg_spec():
    return pl.BlockSpec((2, R, 1), lambda i: (0, i, 0))


def _dvec(h_blk):
    return lax.rsqrt(h_blk[0] + h_blk[1] + 1.0)


_DNUMS = (((1,), (1,)), ((), ()))


def _matmul1(x, w1, b1, deg):
    def kern(x_r, w_r, b_r, h_r, xl_r, y0_r, y1_r):
        xl = lax.dot_general(x_r[...], w_r[...], _DNUMS,
                             preferred_element_type=jnp.float32) + b_r[...]
        xl_r[...] = xl
        y = _dvec(h_r[...]) * xl
        y0_r[...] = y[:, :H2]
        y1_r[...] = y[:, H2:]

    return pl.pallas_call(
        kern,
        grid=(G,),
        in_specs=[
            pl.BlockSpec((R, IN_C), lambda i: (i, 0)),
            pl.BlockSpec((HID, IN_C), lambda i: (0, 0)),
            pl.BlockSpec((1, HID), lambda i: (0, 0)),
            _deg_spec(),
        ],
        out_specs=[
            pl.BlockSpec((R, HID), lambda i: (i, 0)),
            pl.BlockSpec((R, H2), lambda i: (i, 0)),
            pl.BlockSpec((R, H2), lambda i: (i, 0)),
        ],
        out_shape=[
            jax.ShapeDtypeStruct((N, HID), jnp.float32),
            jax.ShapeDtypeStruct((NP, H2), jnp.float32),
            jax.ShapeDtypeStruct((NP, H2), jnp.float32),
        ],
    )(x, w1, b1, deg)


def _combine_mid(a0, a1, xl, deg, m):
    def kern(a0_r, a1_r, xl_r, h_r, m_r, y0_r, y1_r):
        dd = _dvec(h_r[...])
        mm = m_r[...]
        xl = xl_r[...]
        y0_r[...] = dd * jnp.where(mm, dd * a0_r[...], xl[:, :H2])
        y1_r[...] = dd * jnp.where(mm, dd * a1_r[...], xl[:, H2:])

    return pl.pallas_call(
        kern,
        grid=(G,),
        in_specs=[
            pl.BlockSpec((R, H2), lambda i: (i, 0)),
            pl.BlockSpec((R, H2), lambda i: (i, 0)),
            pl.BlockSpec((R, HID), lambda i: (i, 0)),
            _deg_spec(),
            pl.BlockSpec((R, 1), lambda i: (i, 0)),
        ],
        out_specs=[
            pl.BlockSpec((R, H2), lambda i: (i, 0)),
            pl.BlockSpec((R, H2), lambda i: (i, 0)),
        ],
        out_shape=[
            jax.ShapeDtypeStruct((NP, H2), jnp.float32),
            jax.ShapeDtypeStruct((NP, H2), jnp.float32),
        ],
    )(a0, a1, xl, deg, m)


def _layer2_head(a0, a1, xl1, deg, m, bias1, w2, b2):
    def kern(a0_r, a1_r, xl_r, h_r, m_r, b_r, w_r, b2_r, xl2_r, z0_r, z1_r):
        dd = _dvec(h_r[...])
        mm = m_r[...]
        xl = xl_r[...]
        b = b_r[...]
        t0 = jnp.where(mm, xl[:, :H2], dd * a0_r[...]) + b[:, :H2]
        t1 = jnp.where(mm, xl[:, H2:], dd * a1_r[...]) + b[:, H2:]
        t0 = jnp.maximum(t0, 0.0)
        t1 = jnp.maximum(t1, 0.0)
        w = w_r[...]
        xl2 = (lax.dot_general(t0, w[:, :H2], _DNUMS,
                               preferred_element_type=jnp.float32)
               + lax.dot_general(t1, w[:, H2:], _DNUMS,
                                 preferred_element_type=jnp.float32)
               + b2_r[...])
        xl2_r[...] = xl2
        y = dd * xl2
        z0_r[...] = y[:, :H2]
        z1_r[...] = y[:, H2:]

    return pl.pallas_call(
        kern,
        grid=(G,),
        in_specs=[
            pl.BlockSpec((R, H2), lambda i: (i, 0)),
            pl.BlockSpec((R, H2), lambda i: (i, 0)),
            pl.BlockSpec((R, HID), lambda i: (i, 0)),
            _deg_spec(),
            pl.BlockSpec((R, 1), lambda i: (i, 0)),
            pl.BlockSpec((1, HID), lambda i: (0, 0)),
            pl.BlockSpec((HID, HID), lambda i: (0, 0)),
            pl.BlockSpec((1, HID), lambda i: (0, 0)),
        ],
        out_specs=[
            pl.BlockSpec((R, HID), lambda i: (i, 0)),
            pl.BlockSpec((R, H2), lambda i: (i, 0)),
            pl.BlockSpec((R, H2), lambda i: (i, 0)),
        ],
        out_shape=[
            jax.ShapeDtypeStruct((N, HID), jnp.float32),
            jax.ShapeDtypeStruct((NP, H2), jnp.float32),
            jax.ShapeDtypeStruct((NP, H2), jnp.float32),
        ],
    )(a0, a1, xl1, deg, m, bias1, w2, b2)


def _combine_final(a0, a1, xl, deg, m, bias):
    def kern(a0_r, a1_r, xl_r, h_r, m_r, b_r, o_r):
        dd = _dvec(h_r[...])
        mm = m_r[...]
        xl = xl_r[...]
        b = b_r[...]
        t0 = jnp.where(mm, xl[:, :H2], dd * a0_r[...]) + b[:, :H2]
        t1 = jnp.where(mm, xl[:, H2:], dd * a1_r[...]) + b[:, H2:]
        o_r[...] = jnp.concatenate([t0, t1], axis=1)

    return pl.pallas_call(
        kern,
        grid=(G,),
        in_specs=[
            pl.BlockSpec((R, H2), lambda i: (i, 0)),
            pl.BlockSpec((R, H2), lambda i: (i, 0)),
            pl.BlockSpec((R, HID), lambda i: (i, 0)),
            _deg_spec(),
            pl.BlockSpec((R, 1), lambda i: (i, 0)),
            pl.BlockSpec((1, HID), lambda i: (0, 0)),
        ],
        out_specs=pl.BlockSpec((R, HID), lambda i: (i, 0)),
        out_shape=jax.ShapeDtypeStruct((N, HID), jnp.float32),
    )(a0, a1, xl, deg, m, bias)


def kernel(x, edge_index, M, W1, b1, bias1, W2, b2, bias2):
    e3 = edge_index.reshape(2, E // K, K)
    deg = _hist_call(e3).reshape(2, NP, 1)

    b1r = b1.reshape(1, HID)
    b2r = b2.reshape(1, HID)
    bias1r = bias1.reshape(1, HID)
    bias2r = bias2.reshape(1, HID)

    xl1, y0, y1 = _matmul1(x, W1, b1r, deg)
    a0, a1 = _pass_call(y0, y1, e3)
    y0b, y1b = _combine_mid(a0, a1, xl1, deg, M)
    a0b, a1b = _pass_call(y0b, y1b, e3)

    xl2, z0, z1 = _layer2_head(a0b, a1b, xl1, deg, M, bias1r, W2, b2r)

    c0, c1 = _pass_call(z0, z1, e3)
    z0b, z1b = _combine_mid(c0, c1, xl2, deg, M)
    c0b, c1b = _pass_call(z0b, z1b, e3)
    return _combine_final(c0b, c1b, xl2, deg, M, bias2r)
